# baseline (speedup 1.0000x reference)
import functools

import jax
import jax.numpy as jnp
from jax import lax
from jax.experimental import pallas as pl
from jax.experimental.pallas import tpu as pltpu
from jax.experimental.pallas import tpu_sc as plsc

N_NODES = 10000
N_EDGES = 320000
D = 128

NC = 2
NS = 16
NW = NC * NS

N_PAD = 10240
ROWS_PER_TILE = N_PAD // NS
CHUNK = 128
CHUNKS_PER_W = 80
E_PAD = NW * CHUNKS_PER_W * CHUNK
G = 8
NPAIR = CHUNKS_PER_W // (2 * G)
F32 = jnp.float32

_mesh = plsc.VectorSubcoreMesh(core_axis_name="c", subcore_axis_name="s")


RCHUNKS = N_EDGES // CHUNK
RFULL = RCHUNKS // CHUNKS_PER_W
LAST_N = RCHUNKS - RFULL * CHUNKS_PER_W


@functools.partial(
    pl.kernel,
    out_type=jax.ShapeDtypeStruct((NW, CHUNKS_PER_W, CHUNK), F32),
    mesh=_mesh,
    compiler_params=pltpu.CompilerParams(needs_layout_passes=False),
    scratch_types=[
        pltpu.VMEM((CHUNKS_PER_W * CHUNK,), jnp.int32),
        pltpu.VMEM((CHUNKS_PER_W, CHUNK), F32),
    ],
)
def _deg_kernel(ei_hbm, out_hbm, dst_v, deg_v):
    c = lax.axis_index("c")
    s = lax.axis_index("s")
    wid = c * NS + s
    last = wid == NW - 1

    @pl.when(jnp.logical_not(last))
    def _():
        pltpu.sync_copy(
            ei_hbm.at[1, pl.ds(wid * CHUNKS_PER_W * CHUNK,
                               CHUNKS_PER_W * CHUNK)], dst_v)

    @pl.when(last)
    def _():
        pltpu.sync_copy(
            ei_hbm.at[1, pl.ds(RFULL * CHUNKS_PER_W * CHUNK, LAST_N * CHUNK)],
            dst_v.at[pl.ds(0, LAST_N * CHUNK)])

    zeros16 = jnp.zeros((16,), F32)

    def zero_body(i, _):
        deg_v[i // (CHUNK // 16), pl.ds((i % (CHUNK // 16)) * 16, 16)] = zeros16
        return 0

    lax.fori_loop(0, CHUNKS_PER_W * (CHUNK // 16), zero_body, 0)

    ones16 = jnp.ones((16,), F32)
    n_groups = jnp.where(last, LAST_N, CHUNKS_PER_W) * (CHUNK // 16)

    def acc_body(i, _):
        idx = dst_v[pl.ds(i * 16, 16)]
        plsc.addupdate_scatter(
            deg_v, [lax.shift_right_logical(idx, 7),
                    jnp.bitwise_and(idx, 127)], ones16)
        return 0

    lax.fori_loop(0, n_groups, acc_body, 0)
    pltpu.sync_copy(deg_v, out_hbm.at[wid])


@functools.partial(
    pl.kernel,
    out_type=jax.ShapeDtypeStruct((NC, N_PAD, D), F32),
    mesh=_mesh,
    scratch_types=[
        pltpu.VMEM((G * CHUNK,), jnp.int32),
        pltpu.VMEM((G, CHUNK), jnp.int32),
        pltpu.VMEM((G * CHUNK,), jnp.int32),
        pltpu.VMEM((G, CHUNK), jnp.int32),
        pltpu.VMEM((CHUNK, D), F32),
        pltpu.VMEM((CHUNK, D), F32),
        pltpu.VMEM_SHARED((N_PAD, D), F32),
        pltpu.SemaphoreType.DMA,
        pltpu.SemaphoreType.DMA,
        pltpu.SemaphoreType.DMA,
        pltpu.SemaphoreType.DMA,
    ],
)
def _prop_kernel(ei_hbm, dst_hbm, y_hbm, out_hbm,
                 srca_v, dsta_v, srcb_v, dstb_v, rows0_v, rows1_v, acc,
                 sema, semb, sem0, sem1):
    c = lax.axis_index("c")
    s = lax.axis_index("s")
    wid = c * NS + s
    base = wid * CHUNKS_PER_W
    row0 = s * ROWS_PER_TILE
    last = wid == NW - 1
    npair = jnp.where(last, 1, NPAIR)

    pltpu.async_copy(ei_hbm.at[0, pl.ds(base * CHUNK, G * CHUNK)],
                     srca_v, sema)
    pltpu.async_copy(dst_hbm.at[pl.ds(base, G)], dsta_v, sema)

    @pl.when(c == 0)
    def _():
        pltpu.sync_copy(y_hbm.at[pl.ds(row0, ROWS_PER_TILE)],
                        acc.at[pl.ds(row0, ROWS_PER_TILE)])

    @pl.when(c == 1)
    def _():
        zeros16 = jnp.zeros((16,), F32)

        def zb(i, _):
            rows0_v[i // (D // 16), pl.ds((i % (D // 16)) * 16, 16)] = zeros16
            return 0

        lax.fori_loop(0, CHUNK * (D // 16), zb, 0)
        for t in range(ROWS_PER_TILE // CHUNK):
            pltpu.sync_copy(rows0_v, acc.at[pl.ds(row0 + t * CHUNK, CHUNK)])

    plsc.subcore_barrier()

    rows = [rows0_v, rows1_v]
    sems = [sem0, sem1]
    srcs = [srca_v, srcb_v]
    dsts = [dsta_v, dstb_v]

    def _wait_src(sem, dst):
        pltpu.make_async_copy(ei_hbm.at[0, pl.ds(0, G * CHUNK)],
                              dst, sem).wait()

    def _wait_dst(sem, dst):
        pltpu.make_async_copy(dst_hbm.at[pl.ds(0, G)], dst, sem).wait()

    def body(i, _):
        g0 = 2 * i * G
        _wait_src(sema, srca_v)
        _wait_dst(sema, dsta_v)
        pltpu.async_copy(ei_hbm.at[0, pl.ds((base + g0 + G) * CHUNK,
                                            G * CHUNK)], srcb_v, semb)
        pltpu.async_copy(dst_hbm.at[pl.ds(base + g0 + G, G)], dstb_v, semb)
        pltpu.async_copy(y_hbm.at[srca_v.at[pl.ds(0, CHUNK)]], rows0_v, sem0)
        for half in range(2):
            src_v, dst_v = srcs[half], dsts[half]
            for r in range(G):
                rr = half * G + r
                if r < G - 1:
                    pltpu.async_copy(
                        y_hbm.at[src_v.at[pl.ds((r + 1) * CHUNK, CHUNK)]],
                        rows[(rr + 1) % 2], sems[(rr + 1) % 2])
                elif half == 0:
                    _wait_src(semb, srcb_v)
                    _wait_dst(semb, dstb_v)
                    pltpu.async_copy(
                        y_hbm.at[srcb_v.at[pl.ds(0, CHUNK)]],
                        rows[(rr + 1) % 2], sems[(rr + 1) % 2])
                pltpu.make_async_copy(y_hbm.at[pl.ds(0, CHUNK)],
                                      rows[rr % 2], sems[rr % 2]).wait()
                pltpu.sync_copy(rows[rr % 2], acc.at[dst_v.at[r]], add=True)
            if half == 0:
                @pl.when(i + 1 < npair)
                def _():
                    pltpu.async_copy(
                        ei_hbm.at[0, pl.ds((base + g0 + 2 * G) * CHUNK,
                                           G * CHUNK)], srca_v, sema)
                    pltpu.async_copy(dst_hbm.at[pl.ds(base + g0 + 2 * G, G)],
                                     dsta_v, sema)
        return 0

    lax.fori_loop(0, npair, body, 0)

    @pl.when(last)
    def _():
        t0c = RFULL * CHUNKS_PER_W + 2 * G
        pltpu.sync_copy(ei_hbm.at[0, pl.ds(t0c * CHUNK,
                                           (LAST_N - 2 * G) * CHUNK)],
                        srca_v.at[pl.ds(0, (LAST_N - 2 * G) * CHUNK)])
        pltpu.sync_copy(dst_hbm.at[pl.ds(t0c, LAST_N - 2 * G)],
                        dsta_v.at[pl.ds(0, LAST_N - 2 * G)])
        for t in range(LAST_N - 2 * G):
            pltpu.async_copy(y_hbm.at[srca_v.at[pl.ds(t * CHUNK, CHUNK)]],
                             rows0_v, sem0)
            pltpu.make_async_copy(y_hbm.at[pl.ds(0, CHUNK)],
                                  rows0_v, sem0).wait()
            pltpu.sync_copy(rows0_v, acc.at[dsta_v.at[t]], add=True)

    plsc.subcore_barrier()
    pltpu.sync_copy(acc.at[pl.ds(row0, ROWS_PER_TILE)],
                    out_hbm.at[c, pl.ds(row0, ROWS_PER_TILE)])


_BLK = 1024
_GRID = N_PAD // _BLK
_BLK2 = 1280
_GRID2 = N_PAD // _BLK2


def _tc1_body(degp_ref, x_ref, w1_ref, dinv_ref, y_ref):
    deg = jnp.sum(degp_ref[...], axis=0).reshape(_BLK) + 1.0
    d0 = lax.rsqrt(deg)
    dinv = d0 * (1.5 - 0.5 * deg * d0 * d0)
    xl = lax.dot_general(x_ref[...].astype(jnp.bfloat16),
                         w1_ref[...].astype(jnp.bfloat16),
                         (((1,), (1,)), ((), ())),
                         preferred_element_type=F32)
    y_ref[...] = xl * dinv[:, None]
    dinv_ref[...] = dinv[:, None]


def _tc2_body(p_ref, dinv_ref, b1_ref, w2_ref, y2_ref):
    t = (p_ref[0] + p_ref[1]) * dinv_ref[...] + b1_ref[...]
    h = jnp.where(t >= 0, t, 0.01 * t)
    xl = lax.dot_general(h.astype(jnp.bfloat16),
                         w2_ref[...].astype(jnp.bfloat16),
                         (((1,), (1,)), ((), ())),
                         preferred_element_type=F32)
    y2_ref[...] = xl * dinv_ref[...]


def _tc3_body(q_ref, dinv_ref, b2_ref, wd_ref, bd_ref, out_ref):
    t = (q_ref[0] + q_ref[1]) * dinv_ref[...] + b2_ref[...]
    h = jnp.where(t >= 0, t, 0.01 * t)
    hb = h.astype(jnp.bfloat16).astype(F32)
    wb = wd_ref[...].astype(jnp.bfloat16).astype(F32)
    out_ref[...] = jnp.sum(hb * wb, axis=1, keepdims=True) + bd_ref[0]


def kernel(x, edge_index, W1, b1, W2, b2, Wd, bd):
    ei = edge_index.astype(jnp.int32)
    dst2d = ei[1].reshape(RCHUNKS, CHUNK)

    degp = _deg_kernel(ei)

    dinv, y1 = pl.pallas_call(
        _tc1_body,
        grid=(_GRID,),
        in_specs=[
            pl.BlockSpec((NW, _BLK // CHUNK, CHUNK), lambda i: (0, i, 0)),
            pl.BlockSpec((_BLK, D), lambda i: (i, 0)),
            pl.BlockSpec((D, D), lambda i: (0, 0)),
        ],
        out_specs=[
            pl.BlockSpec((_BLK, 1), lambda i: (i, 0)),
            pl.BlockSpec((_BLK, D), lambda i: (i, 0)),
        ],
        out_shape=[
            jax.ShapeDtypeStruct((N_PAD, 1), F32),
            jax.ShapeDtypeStruct((N_PAD, D), F32),
        ],
    )(degp, x, W1)

    p = _prop_kernel(ei, dst2d, y1)

    y2 = pl.pallas_call(
        _tc2_body,
        grid=(_GRID2,),
        in_specs=[
            pl.BlockSpec((NC, _BLK2, D), lambda i: (0, i, 0)),
            pl.BlockSpec((_BLK2, 1), lambda i: (i, 0)),
            pl.BlockSpec((1, D), lambda i: (0, 0)),
            pl.BlockSpec((D, D), lambda i: (0, 0)),
        ],
        out_specs=pl.BlockSpec((_BLK2, D), lambda i: (i, 0)),
        out_shape=jax.ShapeDtypeStruct((N_PAD, D), F32),
    )(p, dinv, b1.reshape(1, D), W2)

    q = _prop_kernel(ei, dst2d, y2)

    out = pl.pallas_call(
        _tc3_body,
        grid=(_GRID2,),
        in_specs=[
            pl.BlockSpec((NC, _BLK2, D), lambda i: (0, i, 0)),
            pl.BlockSpec((_BLK2, 1), lambda i: (i, 0)),
            pl.BlockSpec((1, D), lambda i: (0, 0)),
            pl.BlockSpec((1, D), lambda i: (0, 0)),
            pl.BlockSpec(memory_space=pltpu.MemorySpace.SMEM),
        ],
        out_specs=pl.BlockSpec((_BLK2, 1), lambda i: (i, 0)),
        out_shape=jax.ShapeDtypeStruct((N_NODES, 1), F32),
    )(q, dinv, b2.reshape(1, D), Wd, bd)

    return out


# speedup vs baseline: 1.0009x; 1.0009x over previous
import functools

import jax
import jax.numpy as jnp
from jax import lax
from jax.experimental import pallas as pl
from jax.experimental.pallas import tpu as pltpu
from jax.experimental.pallas import tpu_sc as plsc

N_NODES = 10000
N_EDGES = 320000
D = 128

NC = 2
NS = 16
NW = NC * NS

N_PAD = 10240
ROWS_PER_TILE = N_PAD // NS
CHUNK = 128
CHUNKS_PER_W = 80
G = 8
NPAIR = CHUNKS_PER_W // (2 * G)
F32 = jnp.float32

_mesh = plsc.VectorSubcoreMesh(core_axis_name="c", subcore_axis_name="s")


RCHUNKS = N_EDGES // CHUNK
RFULL = RCHUNKS // CHUNKS_PER_W
LAST_N = RCHUNKS - RFULL * CHUNKS_PER_W


@functools.partial(
    pl.kernel,
    out_type=jax.ShapeDtypeStruct((NW, CHUNKS_PER_W, CHUNK), F32),
    mesh=_mesh,
    compiler_params=pltpu.CompilerParams(needs_layout_passes=False),
    scratch_types=[
        pltpu.VMEM((CHUNKS_PER_W * CHUNK,), jnp.int32),
        pltpu.VMEM((CHUNKS_PER_W, CHUNK), F32),
    ],
)
def _deg_kernel(ei_hbm, out_hbm, dst_v, deg_v):
    c = lax.axis_index("c")
    s = lax.axis_index("s")
    wid = c * NS + s
    last = wid == NW - 1

    @pl.when(jnp.logical_not(last))
    def _():
        pltpu.sync_copy(
            ei_hbm.at[1, pl.ds(wid * CHUNKS_PER_W * CHUNK,
                               CHUNKS_PER_W * CHUNK)], dst_v)

    @pl.when(last)
    def _():
        pltpu.sync_copy(
            ei_hbm.at[1, pl.ds(RFULL * CHUNKS_PER_W * CHUNK, LAST_N * CHUNK)],
            dst_v.at[pl.ds(0, LAST_N * CHUNK)])

    zeros16 = jnp.zeros((16,), F32)

    def zero_body(i, _):
        deg_v[i // (CHUNK // 16), pl.ds((i % (CHUNK // 16)) * 16, 16)] = zeros16
        return 0

    lax.fori_loop(0, CHUNKS_PER_W * (CHUNK // 16), zero_body, 0)

    ones16 = jnp.ones((16,), F32)
    n_groups = jnp.where(last, LAST_N, CHUNKS_PER_W) * (CHUNK // 16)

    def acc_body(i, _):
        idx = dst_v[pl.ds(i * 16, 16)]
        plsc.addupdate_scatter(
            deg_v, [lax.shift_right_logical(idx, 7),
                    jnp.bitwise_and(idx, 127)], ones16)
        return 0

    lax.fori_loop(0, n_groups, acc_body, 0)
    pltpu.sync_copy(deg_v, out_hbm.at[wid])


@functools.partial(
    pl.kernel,
    out_type=jax.ShapeDtypeStruct((NC, N_PAD, D), F32),
    mesh=_mesh,
    scratch_types=[
        pltpu.VMEM((G * CHUNK,), jnp.int32),
        pltpu.VMEM((G, CHUNK), jnp.int32),
        pltpu.VMEM((G * CHUNK,), jnp.int32),
        pltpu.VMEM((G, CHUNK), jnp.int32),
        pltpu.VMEM((CHUNK, D), F32),
        pltpu.VMEM((CHUNK, D), F32),
        pltpu.VMEM_SHARED((N_PAD, D), F32),
        pltpu.SemaphoreType.DMA,
        pltpu.SemaphoreType.DMA,
        pltpu.SemaphoreType.DMA,
        pltpu.SemaphoreType.DMA,
    ],
)
def _prop_kernel(ei_hbm, dst_hbm, y_hbm, out_hbm,
                 srca_v, dsta_v, srcb_v, dstb_v, rows0_v, rows1_v, acc,
                 sema, semb, sem0, sem1):
    c = lax.axis_index("c")
    s = lax.axis_index("s")
    wid = c * NS + s
    base = wid * CHUNKS_PER_W
    row0 = s * ROWS_PER_TILE
    last = wid == NW - 1
    npair = jnp.where(last, 1, NPAIR)

    pltpu.async_copy(ei_hbm.at[0, pl.ds(base * CHUNK, G * CHUNK)],
                     srca_v, sema)
    pltpu.async_copy(dst_hbm.at[pl.ds(base, G)], dsta_v, sema)

    @pl.when(c == 0)
    def _():
        pltpu.sync_copy(y_hbm.at[pl.ds(row0, ROWS_PER_TILE)],
                        acc.at[pl.ds(row0, ROWS_PER_TILE)])

    @pl.when(c == 1)
    def _():
        zeros16 = jnp.zeros((16,), F32)

        def zb(i, _):
            rows0_v[i // (D // 16), pl.ds((i % (D // 16)) * 16, 16)] = zeros16
            return 0

        lax.fori_loop(0, CHUNK * (D // 16), zb, 0)
        for t in range(ROWS_PER_TILE // CHUNK):
            pltpu.sync_copy(rows0_v, acc.at[pl.ds(row0 + t * CHUNK, CHUNK)])

    plsc.subcore_barrier()

    rows = [rows0_v, rows1_v]
    sems = [sem0, sem1]
    srcs = [srca_v, srcb_v]
    dsts = [dsta_v, dstb_v]

    def _wait_src(sem, dst):
        pltpu.make_async_copy(ei_hbm.at[0, pl.ds(0, G * CHUNK)],
                              dst, sem).wait()

    def _wait_dst(sem, dst):
        pltpu.make_async_copy(dst_hbm.at[pl.ds(0, G)], dst, sem).wait()

    def body(i, _):
        g0 = 2 * i * G
        _wait_src(sema, srca_v)
        _wait_dst(sema, dsta_v)
        pltpu.async_copy(ei_hbm.at[0, pl.ds((base + g0 + G) * CHUNK,
                                            G * CHUNK)], srcb_v, semb)
        pltpu.async_copy(dst_hbm.at[pl.ds(base + g0 + G, G)], dstb_v, semb)
        pltpu.async_copy(y_hbm.at[srca_v.at[pl.ds(0, CHUNK)]], rows0_v, sem0)
        for half in range(2):
            src_v, dst_v = srcs[half], dsts[half]
            for r in range(G):
                rr = half * G + r
                if r < G - 1:
                    pltpu.async_copy(
                        y_hbm.at[src_v.at[pl.ds((r + 1) * CHUNK, CHUNK)]],
                        rows[(rr + 1) % 2], sems[(rr + 1) % 2])
                elif half == 0:
                    _wait_src(semb, srcb_v)
                    _wait_dst(semb, dstb_v)
                    pltpu.async_copy(
                        y_hbm.at[srcb_v.at[pl.ds(0, CHUNK)]],
                        rows[(rr + 1) % 2], sems[(rr + 1) % 2])
                pltpu.make_async_copy(y_hbm.at[pl.ds(0, CHUNK)],
                                      rows[rr % 2], sems[rr % 2]).wait()
                pltpu.sync_copy(rows[rr % 2], acc.at[dst_v.at[r]], add=True)
            if half == 0:
                @pl.when(i + 1 < npair)
                def _():
                    pltpu.async_copy(
                        ei_hbm.at[0, pl.ds((base + g0 + 2 * G) * CHUNK,
                                           G * CHUNK)], srca_v, sema)
                    pltpu.async_copy(dst_hbm.at[pl.ds(base + g0 + 2 * G, G)],
                                     dsta_v, sema)
        return 0

    lax.fori_loop(0, npair, body, 0)

    @pl.when(last)
    def _():
        t0c = RFULL * CHUNKS_PER_W + 2 * G
        pltpu.sync_copy(ei_hbm.at[0, pl.ds(t0c * CHUNK,
                                           (LAST_N - 2 * G) * CHUNK)],
                        srca_v.at[pl.ds(0, (LAST_N - 2 * G) * CHUNK)])
        pltpu.sync_copy(dst_hbm.at[pl.ds(t0c, LAST_N - 2 * G)],
                        dsta_v.at[pl.ds(0, LAST_N - 2 * G)])
        for t in range(LAST_N - 2 * G):
            pltpu.async_copy(y_hbm.at[srca_v.at[pl.ds(t * CHUNK, CHUNK)]],
                             rows0_v, sem0)
            pltpu.make_async_copy(y_hbm.at[pl.ds(0, CHUNK)],
                                  rows0_v, sem0).wait()
            pltpu.sync_copy(rows0_v, acc.at[dsta_v.at[t]], add=True)

    plsc.subcore_barrier()
    pltpu.sync_copy(acc.at[pl.ds(row0, ROWS_PER_TILE)],
                    out_hbm.at[c, pl.ds(row0, ROWS_PER_TILE)])


_BLK = 1024
_GRID = N_PAD // _BLK
_BLK2 = 1280
_GRID2 = N_PAD // _BLK2


def _tc1_body(degp_ref, x_ref, w1_ref, dinv_ref, y_ref):
    deg = jnp.sum(degp_ref[...], axis=0).reshape(_BLK) + 1.0
    d0 = lax.rsqrt(deg)
    dinv = d0 * (1.5 - 0.5 * deg * d0 * d0)
    xl = lax.dot_general(x_ref[...].astype(jnp.bfloat16),
                         w1_ref[...].astype(jnp.bfloat16),
                         (((1,), (1,)), ((), ())),
                         preferred_element_type=F32)
    y_ref[...] = xl * dinv[:, None]
    dinv_ref[...] = dinv[:, None]


def _tc2_body(p_ref, dinv_ref, b1_ref, w2_ref, y2_ref):
    t = (p_ref[0] + p_ref[1]) * dinv_ref[...] + b1_ref[...]
    h = jnp.where(t >= 0, t, 0.01 * t)
    xl = lax.dot_general(h.astype(jnp.bfloat16),
                         w2_ref[...].astype(jnp.bfloat16),
                         (((1,), (1,)), ((), ())),
                         preferred_element_type=F32)
    y2_ref[...] = xl * dinv_ref[...]


def _tc3_body(q_ref, dinv_ref, b2_ref, wd_ref, bd_ref, out_ref):
    t = (q_ref[0] + q_ref[1]) * dinv_ref[...] + b2_ref[...]
    h = jnp.where(t >= 0, t, 0.01 * t)
    hb = h.astype(jnp.bfloat16).astype(F32)
    wb = wd_ref[...].astype(jnp.bfloat16).astype(F32)
    out_ref[...] = jnp.sum(hb * wb, axis=1, keepdims=True) + bd_ref[0]


def kernel(x, edge_index, W1, b1, W2, b2, Wd, bd):
    ei = edge_index.astype(jnp.int32)
    dst2d = ei[1].reshape(RCHUNKS, CHUNK)

    degp = _deg_kernel(ei)

    dinv, y1 = pl.pallas_call(
        _tc1_body,
        grid=(_GRID,),
        in_specs=[
            pl.BlockSpec((NW, _BLK // CHUNK, CHUNK), lambda i: (0, i, 0)),
            pl.BlockSpec((_BLK, D), lambda i: (i, 0)),
            pl.BlockSpec((D, D), lambda i: (0, 0)),
        ],
        out_specs=[
            pl.BlockSpec((_BLK, 1), lambda i: (i, 0)),
            pl.BlockSpec((_BLK, D), lambda i: (i, 0)),
        ],
        out_shape=[
            jax.ShapeDtypeStruct((N_PAD, 1), F32),
            jax.ShapeDtypeStruct((N_PAD, D), F32),
        ],
    )(degp, x, W1)

    p = _prop_kernel(ei, dst2d, y1)

    y2 = pl.pallas_call(
        _tc2_body,
        grid=(_GRID2,),
        in_specs=[
            pl.BlockSpec((NC, _BLK2, D), lambda i: (0, i, 0)),
            pl.BlockSpec((_BLK2, 1), lambda i: (i, 0)),
            pl.BlockSpec((1, D), lambda i: (0, 0)),
            pl.BlockSpec((D, D), lambda i: (0, 0)),
        ],
        out_specs=pl.BlockSpec((_BLK2, D), lambda i: (i, 0)),
        out_shape=jax.ShapeDtypeStruct((N_PAD, D), F32),
    )(p, dinv, b1.reshape(1, D), W2)

    q = _prop_kernel(ei, dst2d, y2)

    out = pl.pallas_call(
        _tc3_body,
        grid=(_GRID2,),
        in_specs=[
            pl.BlockSpec((NC, _BLK2, D), lambda i: (0, i, 0)),
            pl.BlockSpec((_BLK2, 1), lambda i: (i, 0)),
            pl.BlockSpec((1, D), lambda i: (0, 0)),
            pl.BlockSpec((1, D), lambda i: (0, 0)),
            pl.BlockSpec(memory_space=pltpu.MemorySpace.SMEM),
        ],
        out_specs=pl.BlockSpec((_BLK2, 1), lambda i: (i, 0)),
        out_shape=jax.ShapeDtypeStruct((N_NODES, 1), F32),
    )(q, dinv, b2.reshape(1, D), Wd, bd)

    return out
